# Initial kernel scaffold; baseline (speedup 1.0000x reference)
import jax
import jax.numpy as jnp
from jax.experimental import pallas as pl


def kernel(x, gate_W, gate_b, W1, b1, W2, b2, gamma, beta):
    raise NotImplementedError("write your pallas kernel here")


# speedup vs baseline: 2.7430x; 2.7430x over previous
import functools

import jax
import jax.numpy as jnp
from jax.experimental import pallas as pl
from jax.experimental.pallas import tpu as pltpu

D = 1024
E = 8
K = 2
FF = 4096
T = 2048
TB = 512
FT = 1024
NF = FF // FT


def _erf(z):
    return jax.lax.erf(z)


def _gelu_exact(v):
    return v * 0.5 * (1.0 + _erf(v * 0.7071067811865476))


def _prep_kernel(x_ref, gw_ref, gb_ref, gamma_ref, beta_ref, xn_ref, w_ref):
    xb = x_ref[...]
    mu = jnp.mean(xb, axis=-1, keepdims=True)
    var = jnp.mean((xb - mu) ** 2, axis=-1, keepdims=True)
    xn = (xb - mu) / jnp.sqrt(var + 1e-5) * gamma_ref[...] + beta_ref[...]
    xn_ref[...] = xn
    logits = jnp.dot(xn, gw_ref[...], preferred_element_type=jnp.float32)
    logits = logits + gb_ref[...]
    m = jnp.max(logits, axis=-1, keepdims=True)
    ex = jnp.exp(logits - m)
    probs = ex / jnp.sum(ex, axis=-1, keepdims=True)
    lane = jax.lax.broadcasted_iota(jnp.int32, probs.shape, 1)
    m0 = jnp.max(probs, axis=-1, keepdims=True)
    e0 = jnp.min(jnp.where(probs == m0, lane, E), axis=-1, keepdims=True)
    probs1 = jnp.where(lane == e0, -1.0, probs)
    m1 = jnp.max(probs1, axis=-1, keepdims=True)
    e1 = jnp.min(jnp.where(probs1 == m1, lane, E), axis=-1, keepdims=True)
    denom = m0 + m1 + 1e-8
    w0 = m0 / denom
    w1 = m1 / denom
    w_ref[...] = jnp.where(lane == e0, w0, 0.0) + jnp.where(lane == e1, w1, 0.0)


def _dense_kernel(x_ref, xn_ref, w_ref, w1_ref, b1_ref, w2_ref, b2_ref,
                  out_ref):
    e = pl.program_id(0)
    f = pl.program_id(1)

    @pl.when(jnp.logical_and(e == 0, f == 0))
    def _init():
        out_ref[...] = x_ref[...]

    xn = xn_ref[...].astype(jnp.bfloat16)
    h = jnp.dot(xn, w1_ref[0].astype(jnp.bfloat16),
                preferred_element_type=jnp.float32) + b1_ref[0]
    h = _gelu_exact(h).astype(jnp.bfloat16)
    y = jnp.dot(h, w2_ref[0].astype(jnp.bfloat16),
                preferred_element_type=jnp.float32)

    lane = jax.lax.broadcasted_iota(jnp.int32, (1, E), 1)
    onehot = (lane == e).astype(jnp.float32)
    wcol = jnp.sum(w_ref[...] * onehot, axis=-1, keepdims=True)

    @pl.when(f == 0)
    def _bias():
        out_ref[...] += wcol * b2_ref[0]

    out_ref[...] += wcol * y


def kernel(x, gate_W, gate_b, W1, b1, W2, b2, gamma, beta):
    b, s, d = x.shape
    flat = x.reshape(-1, d)

    xn, w = pl.pallas_call(
        _prep_kernel,
        grid=(T // TB,),
        in_specs=[
            pl.BlockSpec((TB, D), lambda i: (i, 0)),
            pl.BlockSpec((D, E), lambda i: (0, 0)),
            pl.BlockSpec((E,), lambda i: (0,)),
            pl.BlockSpec((D,), lambda i: (0,)),
            pl.BlockSpec((D,), lambda i: (0,)),
        ],
        out_specs=[
            pl.BlockSpec((TB, D), lambda i: (i, 0)),
            pl.BlockSpec((TB, E), lambda i: (i, 0)),
        ],
        out_shape=[
            jax.ShapeDtypeStruct((T, D), jnp.float32),
            jax.ShapeDtypeStruct((T, E), jnp.float32),
        ],
    )(flat, gate_W, gate_b, gamma, beta)

    out = pl.pallas_call(
        _dense_kernel,
        grid=(E, NF),
        in_specs=[
            pl.BlockSpec((T, D), lambda e, f: (0, 0)),
            pl.BlockSpec((T, D), lambda e, f: (0, 0)),
            pl.BlockSpec((T, E), lambda e, f: (0, 0)),
            pl.BlockSpec((1, D, FT), lambda e, f: (e, 0, f)),
            pl.BlockSpec((1, 1, FT), lambda e, f: (e * NF + f, 0, 0)),
            pl.BlockSpec((1, FT, D), lambda e, f: (e, f, 0)),
            pl.BlockSpec((1, 1, D), lambda e, f: (e, 0, 0)),
        ],
        out_specs=pl.BlockSpec((T, D), lambda e, f: (0, 0)),
        out_shape=jax.ShapeDtypeStruct((T, D), jnp.float32),
        compiler_params=pltpu.CompilerParams(
            dimension_semantics=("arbitrary", "arbitrary"),
        ),
    )(flat, xn, w, W1, b1.reshape(E * NF, 1, FT), W2, b2.reshape(E, 1, D))

    return out.reshape(b, s, d)
